# baseline (speedup 1.0000x reference)
import functools
import math

import jax
import jax.numpy as jnp
from jax import lax
from jax.experimental import pallas as pl
from jax.experimental.pallas import tpu as pltpu
from jax.experimental.pallas import tpu_sc as plsc

D_MODEL = 64
SCALE = math.sqrt(D_MODEL)

_info = plsc.get_sparse_core_info()
NC, NS, L = _info.num_cores, _info.num_subcores, _info.num_lanes
NW = NC * NS


def _make_kernel(B, D, C, NBUF, U):
    per_w = B // NW
    nchunks = per_w // C
    ngroups = nchunks // NBUF
    assert per_w % C == 0 and nchunks % NBUF == 0 and C % U == 0
    assert (C * D) % 128 == 0
    mesh = plsc.VectorSubcoreMesh(core_axis_name="c", subcore_axis_name="s")

    @functools.partial(
        pl.kernel,
        mesh=mesh,
        out_type=jax.ShapeDtypeStruct((B, 2 * D), jnp.float32),
        scratch_types=[
            pltpu.VMEM((NBUF, C), jnp.int32),
            pltpu.VMEM((NBUF, C, D), jnp.float32),
        ]
        + [pltpu.SemaphoreType.DMA] * (2 * NBUF),
        compiler_params=pltpu.CompilerParams(use_tc_tiling_on_sc=False),
    )
    def k(idx_hbm, lut_hbm, out_hbm, idx_v, rows_v, *sems):
        gsem, osem = sems[:NBUF], sems[NBUF:]
        wid = lax.axis_index("s") * NC + lax.axis_index("c")
        base = wid * per_w

        def out_slice(g):
            return out_hbm.at[pl.ds(base + g * C, C), pl.ds(0, D)]

        def load_idx(b, row0):
            pltpu.sync_copy(idx_hbm.at[pl.ds(row0, C)], idx_v.at[b])

        for b in range(NBUF):
            load_idx(b, base + b * C)
            pltpu.async_copy(lut_hbm.at[idx_v.at[b]], rows_v.at[b], gsem[b])

        def group(gi, carry):
            for b in range(NBUF):
                g = gi * NBUF + b
                row0 = base + g * C
                pltpu.make_async_copy(
                    lut_hbm.at[idx_v.at[b]], rows_v.at[b], gsem[b]
                ).wait()
                pltpu.async_copy(rows_v.at[b], out_slice(g), osem[b])
                load_idx(b, row0 + NBUF * C)
                pltpu.make_async_copy(
                    rows_v.at[b], out_slice(g), osem[b]
                ).wait()
                pltpu.async_copy(lut_hbm.at[idx_v.at[b]], rows_v.at[b], gsem[b])
            return carry

        lax.fori_loop(0, ngroups - 1, group, 0)

        for b in range(NBUF):
            g = (ngroups - 1) * NBUF + b
            pltpu.make_async_copy(
                lut_hbm.at[idx_v.at[b]], rows_v.at[b], gsem[b]
            ).wait()
            pltpu.async_copy(rows_v.at[b], out_slice(g), osem[b])
        for b in range(NBUF):
            g = (ngroups - 1) * NBUF + b
            pltpu.make_async_copy(rows_v.at[b], out_slice(g), osem[b]).wait()

    return k


def kernel(x, lut):
    B = x.shape[0] * x.shape[1]
    flat_idx2 = x.reshape(B) * 2
    lut_p = (jnp.pad(lut, ((0, 0), (0, 128 - D_MODEL))) * SCALE).reshape(-1, D_MODEL)
    out128 = _make_kernel(B, D_MODEL, 800, 2, 8)(flat_idx2, lut_p)
    return out128[:, :D_MODEL].reshape(x.shape[0], x.shape[1], D_MODEL)


# speedup vs baseline: 1.0023x; 1.0023x over previous
import functools
import math

import jax
import jax.numpy as jnp
from jax import lax
from jax.experimental import pallas as pl
from jax.experimental.pallas import tpu as pltpu
from jax.experimental.pallas import tpu_sc as plsc

D_MODEL = 64
SCALE = math.sqrt(D_MODEL)

_info = plsc.get_sparse_core_info()
NC, NS, L = _info.num_cores, _info.num_subcores, _info.num_lanes
NW = NC * NS


def _make_kernel(B, D, C, NBUF, U):
    per_w = B // NW
    nchunks = per_w // C
    ngroups = nchunks // NBUF
    assert per_w % C == 0 and nchunks % NBUF == 0 and C % U == 0
    assert (C * D) % 128 == 0
    mesh = plsc.VectorSubcoreMesh(core_axis_name="c", subcore_axis_name="s")

    @functools.partial(
        pl.kernel,
        mesh=mesh,
        out_type=jax.ShapeDtypeStruct((B, 2 * D), jnp.float32),
        scratch_types=[
            pltpu.VMEM((NBUF, C), jnp.int32),
            pltpu.VMEM((NBUF, C, D), jnp.float32),
        ]
        + [pltpu.SemaphoreType.DMA] * (2 * NBUF),
        compiler_params=pltpu.CompilerParams(use_tc_tiling_on_sc=False),
    )
    def k(idx_hbm, lut_hbm, out_hbm, idx_v, rows_v, *sems):
        gsem, osem = sems[:NBUF], sems[NBUF:]
        wid = lax.axis_index("s") * NC + lax.axis_index("c")
        base = wid * per_w

        def out_slice(g):
            return out_hbm.at[pl.ds(base + g * C, C), pl.ds(0, D)]

        def load_idx(b, row0):
            pltpu.sync_copy(idx_hbm.at[pl.ds(row0, C)], idx_v.at[b])

        for b in range(NBUF):
            load_idx(b, base + b * C)
            pltpu.async_copy(lut_hbm.at[idx_v.at[b]], rows_v.at[b], gsem[b])

        def group(gi, carry):
            for b in range(NBUF):
                g = gi * NBUF + b
                row0 = base + g * C
                pltpu.make_async_copy(
                    lut_hbm.at[idx_v.at[b]], rows_v.at[b], gsem[b]
                ).wait()
                pltpu.async_copy(rows_v.at[b], out_slice(g), osem[b])
                load_idx(b, row0 + NBUF * C)
                pltpu.make_async_copy(
                    rows_v.at[b], out_slice(g), osem[b]
                ).wait()
                pltpu.async_copy(lut_hbm.at[idx_v.at[b]], rows_v.at[b], gsem[b])
            return carry

        lax.fori_loop(0, ngroups - 1, group, 0)

        for b in range(NBUF):
            g = (ngroups - 1) * NBUF + b
            pltpu.make_async_copy(
                lut_hbm.at[idx_v.at[b]], rows_v.at[b], gsem[b]
            ).wait()
            pltpu.async_copy(rows_v.at[b], out_slice(g), osem[b])
        for b in range(NBUF):
            g = (ngroups - 1) * NBUF + b
            pltpu.make_async_copy(rows_v.at[b], out_slice(g), osem[b]).wait()

    return k


def kernel(x, lut):
    B = x.shape[0] * x.shape[1]
    flat_idx2 = x.reshape(B) * 2
    lut_p = (jnp.pad(lut, ((0, 0), (0, 128 - D_MODEL))) * SCALE).reshape(-1, D_MODEL)
    out128 = _make_kernel(B, D_MODEL, 400, 4, 8)(flat_idx2, lut_p)
    return out128[:, :D_MODEL].reshape(x.shape[0], x.shape[1], D_MODEL)


# speedup vs baseline: 1.0024x; 1.0001x over previous
import functools
import math

import jax
import jax.numpy as jnp
from jax import lax
from jax.experimental import pallas as pl
from jax.experimental.pallas import tpu as pltpu
from jax.experimental.pallas import tpu_sc as plsc

D_MODEL = 64
SCALE = math.sqrt(D_MODEL)

_info = plsc.get_sparse_core_info()
NC, NS, L = _info.num_cores, _info.num_subcores, _info.num_lanes
NW = NC * NS


def _make_kernel(B, D, C, NBUF):
    per_w = B // NW
    nchunks = per_w // C
    ngroups = nchunks // NBUF
    assert per_w % C == 0 and nchunks % NBUF == 0
    mesh = plsc.VectorSubcoreMesh(core_axis_name="c", subcore_axis_name="s")

    @functools.partial(
        pl.kernel,
        mesh=mesh,
        out_type=jax.ShapeDtypeStruct((B, 2 * D), jnp.float32),
        scratch_types=[
            pltpu.VMEM((NBUF, C), jnp.int32),
            pltpu.VMEM((NBUF, C, D), jnp.float32),
        ]
        + [pltpu.SemaphoreType.DMA] * (2 * NBUF),
        compiler_params=pltpu.CompilerParams(use_tc_tiling_on_sc=False),
    )
    def k(idx_hbm, lut_hbm, out_hbm, idx_v, rows_v, *sems):
        gsem, osem = sems[:NBUF], sems[NBUF:]
        wid = lax.axis_index("s") * NC + lax.axis_index("c")
        base = wid * per_w

        def out_slice(g):
            return out_hbm.at[pl.ds(base + g * C, C), pl.ds(0, D)]

        def load_idx(b, row0):
            pltpu.sync_copy(idx_hbm.at[pl.ds(row0, C)], idx_v.at[b])

        for b in range(NBUF):
            load_idx(b, base + b * C)
            pltpu.async_copy(lut_hbm.at[idx_v.at[b]], rows_v.at[b], gsem[b])

        def group(gi, carry):
            for b in range(NBUF):
                g = gi * NBUF + b
                row0 = base + g * C
                pltpu.make_async_copy(
                    lut_hbm.at[idx_v.at[b]], rows_v.at[b], gsem[b]
                ).wait()
                pltpu.async_copy(rows_v.at[b], out_slice(g), osem[b])
                load_idx(b, row0 + NBUF * C)
                pltpu.make_async_copy(
                    rows_v.at[b], out_slice(g), osem[b]
                ).wait()
                pltpu.async_copy(lut_hbm.at[idx_v.at[b]], rows_v.at[b], gsem[b])
            return carry

        lax.fori_loop(0, ngroups - 1, group, 0)

        for b in range(NBUF):
            g = (ngroups - 1) * NBUF + b
            pltpu.make_async_copy(
                lut_hbm.at[idx_v.at[b]], rows_v.at[b], gsem[b]
            ).wait()
            pltpu.async_copy(rows_v.at[b], out_slice(g), osem[b])
        for b in range(NBUF):
            g = (ngroups - 1) * NBUF + b
            pltpu.make_async_copy(rows_v.at[b], out_slice(g), osem[b]).wait()

    return k


def kernel(x, lut):
    B = x.shape[0] * x.shape[1]
    flat_idx2 = x.reshape(B) * 2
    lut_p = (jnp.pad(lut, ((0, 0), (0, 128 - D_MODEL))) * SCALE).reshape(-1, D_MODEL)
    out128 = _make_kernel(B, D_MODEL, 400, 4)(flat_idx2, lut_p)
    return out128[:, :D_MODEL].reshape(x.shape[0], x.shape[1], D_MODEL)
